# baseline (speedup 1.0000x reference)
import functools

import jax
import jax.numpy as jnp
from jax import lax
from jax.experimental import pallas as pl
from jax.experimental.pallas import tpu as pltpu
from jax.experimental.pallas import tpu_sc as plsc

NC = 2
NS = 16
NW = NC * NS
NBUF = 8
K = 4
LANE = 128
PADH = 56

XBLK = 2048
OBLK = 256


def _fmt_x(x):
    batch, hist = x.shape

    def body(x_ref, o_ref):
        o_ref[...] = jnp.pad(x_ref[...], ((0, 0), (0, LANE - hist)))

    return pl.pallas_call(
        body,
        grid=(batch // XBLK,),
        in_specs=[pl.BlockSpec((XBLK, hist), lambda i: (i, 0))],
        out_specs=pl.BlockSpec((XBLK, LANE), lambda i: (i, 0)),
        out_shape=jax.ShapeDtypeStruct((batch, LANE), jnp.int32),
    )(x)


def _fmt_out(ypad, batch, hist, d):
    def body(y_ref, o_ref):
        o_ref[...] = y_ref[...].reshape(OBLK, PADH, d)[:, :hist, :]

    return pl.pallas_call(
        body,
        grid=(batch // OBLK,),
        in_specs=[pl.BlockSpec((OBLK * PADH, d), lambda i: (i, 0))],
        out_specs=pl.BlockSpec((OBLK, hist, d), lambda i: (i, 0, 0)),
        out_shape=jax.ShapeDtypeStruct((batch, hist, d), jnp.float32),
    )(ypad)


def _make_gather_kernel(batch: int, hist: int, d: int):
    rows_per_w = batch // NW
    assert batch % NW == 0 and rows_per_w % NBUF == 0

    mesh = plsc.VectorSubcoreMesh(
        core_axis_name="c", subcore_axis_name="s",
        num_cores=NC, num_subcores=NS,
    )

    @functools.partial(
        pl.kernel,
        out_type=jax.ShapeDtypeStruct((batch * PADH, d), jnp.float32),
        mesh=mesh,
        scratch_types=(
            pltpu.VMEM((rows_per_w, LANE), jnp.int32),
            [pltpu.VMEM((PADH, d), jnp.float32) for _ in range(NBUF)],
            [pltpu.SemaphoreType.DMA for _ in range(NBUF)],
            [pltpu.SemaphoreType.DMA for _ in range(NBUF)],
        ),
        compiler_params=pltpu.CompilerParams(use_tc_tiling_on_sc=False),
    )
    def gather(idx_hbm, table_hbm, out_hbm, idx_v, rows, gsem, wsem):
        wid = lax.axis_index("s") * NC + lax.axis_index("c")
        base = wid * rows_per_w
        pltpu.sync_copy(idx_hbm.at[pl.ds(base, rows_per_w)], idx_v)

        def fire(r, b):
            pltpu.async_copy(
                table_hbm.at[idx_v.at[r, pl.ds(0, PADH)]], rows[b], gsem[b]
            )

        def put(r, b):
            pltpu.async_copy(
                rows[b],
                out_hbm.at[pl.ds((base + r) * PADH, PADH)],
                wsem[b],
            )

        def wait_put(b):
            pltpu.make_async_copy(
                rows[b],
                out_hbm.at[pl.ds(base * PADH, PADH)],
                wsem[b],
            ).wait()

        for jj in range(K):
            fire(jj, jj)

        def step(i, _):
            for b in range(NBUF):
                j = i * NBUF + b
                jk = j + K
                bk = (b + K) % NBUF

                @pl.when(jnp.logical_and(jk >= NBUF, jk < rows_per_w))
                def _():
                    wait_put(bk)

                @pl.when(jk < rows_per_w)
                def _():
                    fire(jk, bk)

                pltpu.make_async_copy(
                    table_hbm.at[idx_v.at[b, pl.ds(0, PADH)]], rows[b], gsem[b]
                ).wait()
                put(j, b)

            return 0

        lax.fori_loop(0, rows_per_w // NBUF, step, 0)

        for b in range(NBUF):
            wait_put(b)

    return gather


def kernel(x, weight):
    b, h = x.shape
    d = weight.shape[1]
    xi = _fmt_x(x.astype(jnp.int32))
    ypad = _make_gather_kernel(b, h, d)(xi, weight)
    return _fmt_out(ypad, b, h, d)


# speedup vs baseline: 3.0877x; 3.0877x over previous
import functools

import jax
import jax.numpy as jnp
from jax import lax
from jax.experimental import pallas as pl
from jax.experimental.pallas import tpu as pltpu
from jax.experimental.pallas import tpu_sc as plsc

NC = 2
NS = 16
NW = NC * NS
NBUF = 8
K = 4


def _make_embed_kernel(batch: int, hist: int, d: int):
    rows_per_w = batch // NW
    assert batch % NW == 0 and rows_per_w % NBUF == 0

    mesh = plsc.VectorSubcoreMesh(
        core_axis_name="c", subcore_axis_name="s",
        num_cores=NC, num_subcores=NS,
    )

    @functools.partial(
        pl.kernel,
        out_type=jax.ShapeDtypeStruct((batch, hist, d), jnp.float32),
        mesh=mesh,
        scratch_types=(
            pltpu.VMEM((rows_per_w, hist), jnp.int32),
            [pltpu.VMEM((hist, d), jnp.float32) for _ in range(NBUF)],
            [pltpu.SemaphoreType.DMA for _ in range(NBUF)],
            [pltpu.SemaphoreType.DMA for _ in range(NBUF)],
        ),
        compiler_params=pltpu.CompilerParams(use_tc_tiling_on_sc=False),
    )
    def embed(idx_hbm, table_hbm, out_hbm, idx_v, rows, gsem, wsem):
        wid = lax.axis_index("s") * NC + lax.axis_index("c")
        base = wid * rows_per_w
        pltpu.sync_copy(idx_hbm.at[pl.ds(base, rows_per_w)], idx_v)

        for jj in range(K):
            pltpu.async_copy(table_hbm.at[idx_v.at[jj]], rows[jj], gsem[jj])

        def step(i, _):
            for b in range(NBUF):
                j = i * NBUF + b
                jk = j + K
                bk = (b + K) % NBUF

                @pl.when(jnp.logical_and(jk >= NBUF, jk < rows_per_w))
                def _():
                    pltpu.make_async_copy(
                        rows[bk], out_hbm.at[base], wsem[bk]
                    ).wait()

                @pl.when(jk < rows_per_w)
                def _():
                    pltpu.async_copy(
                        table_hbm.at[idx_v.at[jk]], rows[bk], gsem[bk]
                    )

                pltpu.make_async_copy(
                    table_hbm.at[idx_v.at[b]], rows[b], gsem[b]
                ).wait()
                pltpu.async_copy(rows[b], out_hbm.at[base + j], wsem[b])

            return 0

        lax.fori_loop(0, rows_per_w // NBUF, step, 0)

        for b in range(NBUF):
            pltpu.make_async_copy(
                rows[b], out_hbm.at[base], wsem[b]
            ).wait()

    return embed


def kernel(x, weight):
    b, h = x.shape
    return _make_embed_kernel(b, h, weight.shape[1])(
        x.astype(jnp.int32), weight
    )


# speedup vs baseline: 3.1019x; 1.0046x over previous
import functools

import jax
import jax.numpy as jnp
from jax import lax
from jax.experimental import pallas as pl
from jax.experimental.pallas import tpu as pltpu
from jax.experimental.pallas import tpu_sc as plsc

N_ROWS = 1000000
D = 64
NC = 2
NS = 16
NW = NC * NS
CHUNK = 128
NBUF = 8
K = 4


def _make_embed_kernel(n_idx: int):
    per_w = n_idx // NW
    n_chunks = per_w // CHUNK
    assert per_w % CHUNK == 0 and n_chunks % NBUF == 0

    mesh = plsc.VectorSubcoreMesh(
        core_axis_name="c", subcore_axis_name="s",
        num_cores=NC, num_subcores=NS,
    )

    @functools.partial(
        pl.kernel,
        out_type=jax.ShapeDtypeStruct((n_idx, D), jnp.float32),
        mesh=mesh,
        scratch_types=(
            pltpu.VMEM((n_chunks, CHUNK), jnp.int32),
            [pltpu.VMEM((CHUNK, D), jnp.float32) for _ in range(NBUF)],
            [pltpu.SemaphoreType.DMA for _ in range(NBUF)],
            [pltpu.SemaphoreType.DMA for _ in range(NBUF)],
        ),
        compiler_params=pltpu.CompilerParams(use_tc_tiling_on_sc=False),
    )
    def embed(idx_hbm, table_hbm, out_hbm, idx_v, rows, gsem, wsem):
        wid = lax.axis_index("s") * NC + lax.axis_index("c")
        base = wid * per_w
        pltpu.sync_copy(idx_hbm.at[wid], idx_v)

        for jj in range(K):
            pltpu.async_copy(table_hbm.at[idx_v.at[jj]], rows[jj], gsem[jj])

        def step(i, _):
            for b in range(NBUF):
                j = i * NBUF + b
                jk = j + K
                bk = (b + K) % NBUF

                @pl.when(jnp.logical_and(jk >= NBUF, jk < n_chunks))
                def _():
                    pltpu.make_async_copy(
                        rows[bk], out_hbm.at[pl.ds(base, CHUNK)], wsem[bk]
                    ).wait()

                @pl.when(jk < n_chunks)
                def _():
                    pltpu.async_copy(table_hbm.at[idx_v.at[jk]], rows[bk], gsem[bk])

                pltpu.make_async_copy(
                    table_hbm.at[idx_v.at[b]], rows[b], gsem[b]
                ).wait()
                pltpu.async_copy(
                    rows[b], out_hbm.at[pl.ds(base + j * CHUNK, CHUNK)], wsem[b]
                )

            return 0

        lax.fori_loop(0, n_chunks // NBUF, step, 0)

        for b in range(NBUF):
            pltpu.make_async_copy(
                rows[b], out_hbm.at[pl.ds(base, CHUNK)], wsem[b]
            ).wait()

    return embed


def kernel(x, weight):
    b, h = x.shape
    n_idx = b * h
    idx = x.reshape(NW, (n_idx // NW) // CHUNK, CHUNK).astype(jnp.int32)
    out = _make_embed_kernel(n_idx)(idx, weight)
    return out.reshape(b, h, weight.shape[1])
